# baseline (speedup 1.0000x reference)
import functools

import jax
import jax.numpy as jnp
from jax import lax
from jax.experimental import pallas as pl
from jax.experimental.pallas import tpu as pltpu
from jax.experimental.pallas import tpu_sc as plsc


_INFO = plsc.get_sparse_core_info()
_NC = _INFO.num_cores
_NS = _INFO.num_subcores
_NW = _NC * _NS
_CH = 128


@functools.partial(jax.jit, static_argnums=(2, 3))
def _sc_gather(idx_flat, table, nch, hidden):
    n_rows = _NW * nch * _CH
    mesh = plsc.VectorSubcoreMesh(core_axis_name="c", subcore_axis_name="s")

    nbuf = 2
    vocab = table.shape[0]

    @functools.partial(
        pl.kernel,
        out_type=jax.ShapeDtypeStruct((n_rows // _CH, _CH, hidden), jnp.float32),
        mesh=mesh,
        scratch_types=[
            pltpu.VMEM((nch * _CH,), jnp.int32),
            [pltpu.VMEM((1, _CH, hidden), jnp.float32)] * nbuf,
            pltpu.VMEM_SHARED((vocab, hidden), jnp.float32),
            [pltpu.SemaphoreType.DMA] * nbuf,
            [pltpu.SemaphoreType.DMA] * nbuf,
        ],
    )
    def body(idx_hbm, table_hbm, out_hbm, idx_v, bufs, tab_sh, gsems, ssems):
        wid = lax.axis_index("s") * _NC + lax.axis_index("c")
        base = wid * nch
        sid = lax.axis_index("s")

        @pl.when(sid == 0)
        def _stage_table():
            pltpu.sync_copy(table_hbm, tab_sh)

        pltpu.sync_copy(idx_hbm.at[pl.ds(wid * nch * _CH, nch * _CH)], idx_v)
        plsc.subcore_barrier()

        def loop_body(i, _):
            c = i * 2
            g0 = pltpu.async_copy(
                tab_sh.at[idx_v.at[pl.ds(c * _CH, _CH)]], bufs[0].at[0], gsems[0]
            )
            g1 = pltpu.async_copy(
                tab_sh.at[idx_v.at[pl.ds((c + 1) * _CH, _CH)]], bufs[1].at[0], gsems[1]
            )
            g0.wait()
            s0 = pltpu.async_copy(bufs[0], out_hbm.at[pl.ds(base + c, 1)], ssems[0])
            g1.wait()
            s1 = pltpu.async_copy(bufs[1], out_hbm.at[pl.ds(base + c + 1, 1)], ssems[1])
            s0.wait()
            s1.wait()
            return ()

        lax.fori_loop(0, nch // 2, loop_body, ())

    return body(idx_flat, table)


def kernel(input_ids, attention_mask, embed_weight):
    del attention_mask
    batch, seq = input_ids.shape
    vocab, hidden = embed_weight.shape
    n_rows = batch * seq
    nch = n_rows // (_NW * _CH)
    ids = input_ids.reshape(-1).astype(jnp.int32)
    table = embed_weight.astype(jnp.float32)
    out = _sc_gather(ids, table, nch, hidden)
    return out.reshape(batch, seq, hidden)


# speedup vs baseline: 1.0952x; 1.0952x over previous
import functools

import jax
import jax.numpy as jnp
from jax import lax
from jax.experimental import pallas as pl
from jax.experimental.pallas import tpu as pltpu
from jax.experimental.pallas import tpu_sc as plsc


_INFO = plsc.get_sparse_core_info()
_NC = _INFO.num_cores
_NS = _INFO.num_subcores
_NW = _NC * _NS
_CH = 128
_GPB = 2


@functools.partial(jax.jit, static_argnums=(2, 3))
def _sc_gather(ids2d, table, nch, hidden):
    batch, seq = ids2d.shape
    n_rows = batch * seq
    wpb = _NW // batch
    mesh = plsc.VectorSubcoreMesh(core_axis_name="c", subcore_axis_name="s")

    nbuf = 3
    nst = nch // _GPB
    vocab = table.shape[0]

    @functools.partial(
        pl.kernel,
        out_type=jax.ShapeDtypeStruct((n_rows // _CH, _CH, hidden), jnp.float32),
        mesh=mesh,
        scratch_types=[
            pltpu.VMEM((nch * _CH,), jnp.int32),
            [pltpu.VMEM((_GPB, _CH, hidden), jnp.float32)] * nbuf,
            pltpu.VMEM_SHARED((vocab, hidden), jnp.float32),
            [[pltpu.SemaphoreType.DMA] * _GPB] * nbuf,
            [pltpu.SemaphoreType.DMA] * nbuf,
        ],
    )
    def body(idx_hbm, table_hbm, out_hbm, idx_v, bufs, tab_sh, gsems, ssems):
        wid = lax.axis_index("s") * _NC + lax.axis_index("c")
        row = wid // wpb
        col = (wid % wpb) * (nch * _CH)
        base = wid * nch
        sid = lax.axis_index("s")

        @pl.when(sid == 0)
        def _stage_table():
            pltpu.sync_copy(table_hbm, tab_sh)

        pltpu.sync_copy(idx_hbm.at[row, pl.ds(col, nch * _CH)], idx_v)
        plsc.subcore_barrier()

        def gather(st, b):
            return [
                pltpu.async_copy(
                    tab_sh.at[idx_v.at[pl.ds((st * _GPB + j) * _CH, _CH)]],
                    bufs[b].at[j],
                    gsems[b][j],
                )
                for j in range(_GPB)
            ]

        gathers = [None] * nbuf
        stores = [None] * nbuf
        for st in range(min(nbuf, nst)):
            gathers[st] = gather(st, st)
        for st in range(nst):
            b = st % nbuf
            for g in gathers[b]:
                g.wait()
            stores[b] = pltpu.async_copy(
                bufs[b], out_hbm.at[pl.ds(base + st * _GPB, _GPB)], ssems[b]
            )
            nxt = st + nbuf
            if nxt < nst:
                stores[b].wait()
                gathers[b] = gather(nxt, b)
                stores[b] = None
        for s in stores:
            if s is not None:
                s.wait()

    return body(ids2d, table)


def kernel(input_ids, attention_mask, embed_weight):
    del attention_mask
    batch, seq = input_ids.shape
    vocab, hidden = embed_weight.shape
    n_rows = batch * seq
    nch = n_rows // (_NW * _CH)
    ids = input_ids.astype(jnp.int32)
    table = embed_weight.astype(jnp.float32)
    out = _sc_gather(ids, table, nch, hidden)
    return out.reshape(batch, seq, hidden)
